# Initial kernel scaffold; baseline (speedup 1.0000x reference)
import jax
import jax.numpy as jnp
from jax.experimental import pallas as pl


def kernel(points_coords, points_labels, boxes_coords, boxes_labels, pe_gauss, pt_w0, pt_w1, bx_w0, bx_w1, bx_w2, bx_w3, W_up, b_up):
    raise NotImplementedError("write your pallas kernel here")


# speedup vs baseline: 1.3579x; 1.3579x over previous
import functools
import math

import jax
import jax.numpy as jnp
from jax.experimental import pallas as pl
from jax.experimental.pallas import tpu as pltpu

_TWO_PI = 2.0 * math.pi


def _body(px_ref, py_ref, plab_ref,
          b0x_ref, b0y_ref, b1x_ref, b1y_ref, blab_ref,
          gauss_ref, ptw0_ref, ptw1_ref,
          bxw0_ref, bxw1_ref, bxw2_ref, bxw3_ref,
          wup_ref, bup_ref, out_ref, *, np_, nb, half_d):
    def _bf(v):
        return v.astype(jnp.bfloat16).astype(jnp.float32)

    g = gauss_ref[...]
    g0 = _bf(g[0:1, :][None])
    g1 = _bf(g[1:2, :][None])

    def _theta(x, y):
        tx = _bf(2.0 * (x + 0.5) - 1.0)[:, :, None]
        ty = _bf(2.0 * (y + 0.5) - 1.0)[:, :, None]
        return _TWO_PI * (tx * g0 + ty * g1)

    theta = _theta(px_ref[...], py_ref[...])
    emb = jnp.concatenate([jnp.sin(theta), jnp.cos(theta)], axis=-1)

    plab = plab_ref[...]
    m0 = (plab == 0).astype(jnp.float32)[:, :, None]
    m1 = (plab == 1).astype(jnp.float32)[:, :, None]
    emb = emb + m0 * ptw0_ref[...][None] + m1 * ptw1_ref[...][None]

    bb = emb.shape[0]
    d = 2 * half_d
    emb2 = emb.reshape(bb * np_, d)
    pts = jax.lax.dot_general(emb2.astype(jnp.bfloat16),
                              wup_ref[...].astype(jnp.bfloat16),
                              (((1,), (1,)), ((), ())),
                              preferred_element_type=jnp.float32)
    pts = pts + bup_ref[...]
    out_ref[:, 0:np_, :] = pts.reshape(bb, np_, 2 * d)

    def corner(cx_ref, cy_ref, w_ref):
        th = _theta(cx_ref[...], cy_ref[...])
        e = jnp.concatenate([jnp.sin(th), jnp.cos(th)], axis=-1)
        return e + w_ref[...][None]

    e0 = corner(b0x_ref, b0y_ref, bxw2_ref)
    e1 = corner(b1x_ref, b1y_ref, bxw3_ref)
    box = jnp.concatenate([e0, e1], axis=-1)

    blab = blab_ref[...]
    n0 = (blab == 0).astype(jnp.float32)[:, :, None]
    n1 = (blab == 1).astype(jnp.float32)[:, :, None]
    box = box + n0 * bxw0_ref[...][None] + n1 * bxw1_ref[...][None]
    out_ref[:, np_:np_ + nb, :] = box


def kernel(points_coords, points_labels, boxes_coords, boxes_labels,
           pe_gauss, pt_w0, pt_w1, bx_w0, bx_w1, bx_w2, bx_w3, W_up, b_up):
    B, NP, _ = points_coords.shape
    NB = boxes_coords.shape[1]
    HALF_D = pe_gauss.shape[1]
    D = 2 * HALF_D

    bB = 64
    grid = (B // bB,)

    px = points_coords[..., 0]
    py = points_coords[..., 1]
    b0x = boxes_coords[..., 0]
    b0y = boxes_coords[..., 1]
    b1x = boxes_coords[..., 2]
    b1y = boxes_coords[..., 3]
    plab = points_labels.astype(jnp.int32)
    blab = boxes_labels.astype(jnp.int32)
    bup2 = b_up.reshape(1, 2 * D)

    def batch_spec(n):
        return pl.BlockSpec((bB, n), lambda i: (i, 0))

    def full_spec(shape):
        return pl.BlockSpec(shape, lambda i: tuple(0 for _ in shape))

    out = pl.pallas_call(
        functools.partial(_body, np_=NP, nb=NB, half_d=HALF_D),
        grid=grid,
        in_specs=[
            batch_spec(NP), batch_spec(NP), batch_spec(NP),
            batch_spec(NB), batch_spec(NB), batch_spec(NB), batch_spec(NB),
            batch_spec(NB),
            full_spec((2, HALF_D)),
            full_spec((1, D)), full_spec((1, D)),
            full_spec((1, 2 * D)), full_spec((1, 2 * D)),
            full_spec((1, D)), full_spec((1, D)),
            full_spec((2 * D, D)), full_spec((1, 2 * D)),
        ],
        out_specs=pl.BlockSpec((bB, NP + NB, 2 * D), lambda i: (i, 0, 0)),
        out_shape=jax.ShapeDtypeStruct((B, NP + NB, 2 * D), jnp.float32),
        compiler_params=pltpu.CompilerParams(
            dimension_semantics=("arbitrary",),
        ),
    )(px, py, plab, b0x, b0y, b1x, b1y, blab,
      pe_gauss, pt_w0, pt_w1, bx_w0, bx_w1, bx_w2, bx_w3, W_up, bup2)
    return out


# speedup vs baseline: 3.0937x; 2.2783x over previous
import functools
import math

import jax
import jax.numpy as jnp
from jax.experimental import pallas as pl
from jax.experimental.pallas import tpu as pltpu

_RND = 12582912.0

_SIN_C = (6.28318528104831, -41.34169782418005, 81.60504599287985,
          -76.70182092383982, 42.01785862976225, -14.873782832018605,
          3.205829400218412)
_COS_C = (0.9999999999544659, -19.739208759308465, 64.93938990191846,
          -85.45667587093784, 60.24231603222898, -26.405687719149896,
          7.802961250605405, -1.4562273910586767)


def _sincos_2pi(p):
    k = (p + _RND) - _RND
    u = p - k
    w = u * u
    s = _SIN_C[-1]
    for c in _SIN_C[-2::-1]:
        s = s * w + c
    s = s * u
    c_ = _COS_C[-1]
    for c in _COS_C[-2::-1]:
        c_ = c_ * w + c
    return s, c_


def _body(px_ref, py_ref, plab_ref,
          b0x_ref, b0y_ref, b1x_ref, b1y_ref, blab_ref,
          gauss_ref, ptw0_ref, ptw1_ref,
          bxw0_ref, bxw1_ref, bxw2_ref, bxw3_ref,
          wup_ref, bup_ref, out_ref, *, np_, nb, half_d):
    def _bf(v):
        return v.astype(jnp.bfloat16).astype(jnp.float32)

    g = gauss_ref[...]
    g0 = _bf(g[0:1, :][None])
    g1 = _bf(g[1:2, :][None])

    def _proj(x, y):
        tx = _bf(2.0 * (x + 0.5) - 1.0)[:, :, None]
        ty = _bf(2.0 * (y + 0.5) - 1.0)[:, :, None]
        return tx * g0 + ty * g1

    s, c = _sincos_2pi(_proj(px_ref[...], py_ref[...]))
    emb = jnp.concatenate([s, c], axis=-1)

    plab = plab_ref[...]
    m0 = (plab == 0).astype(jnp.float32)[:, :, None]
    m1 = (plab == 1).astype(jnp.float32)[:, :, None]
    emb = emb + m0 * ptw0_ref[...][None] + m1 * ptw1_ref[...][None]

    bb = emb.shape[0]
    d = 2 * half_d
    emb2 = emb.reshape(bb * np_, d)
    pts = jax.lax.dot_general(emb2.astype(jnp.bfloat16),
                              wup_ref[...].astype(jnp.bfloat16),
                              (((1,), (1,)), ((), ())),
                              preferred_element_type=jnp.float32)
    pts = pts + bup_ref[...]
    out_ref[:, 0:np_, :] = pts.reshape(bb, np_, 2 * d)

    def corner(cx_ref, cy_ref, w_ref):
        sc, cc = _sincos_2pi(_proj(cx_ref[...], cy_ref[...]))
        e = jnp.concatenate([sc, cc], axis=-1)
        return e + w_ref[...][None]

    e0 = corner(b0x_ref, b0y_ref, bxw2_ref)
    e1 = corner(b1x_ref, b1y_ref, bxw3_ref)
    box = jnp.concatenate([e0, e1], axis=-1)

    blab = blab_ref[...]
    n0 = (blab == 0).astype(jnp.float32)[:, :, None]
    n1 = (blab == 1).astype(jnp.float32)[:, :, None]
    box = box + n0 * bxw0_ref[...][None] + n1 * bxw1_ref[...][None]
    out_ref[:, np_:np_ + nb, :] = box


def kernel(points_coords, points_labels, boxes_coords, boxes_labels,
           pe_gauss, pt_w0, pt_w1, bx_w0, bx_w1, bx_w2, bx_w3, W_up, b_up):
    B, NP, _ = points_coords.shape
    NB = boxes_coords.shape[1]
    HALF_D = pe_gauss.shape[1]
    D = 2 * HALF_D

    bB = 64
    grid = (B // bB,)

    px = points_coords[..., 0]
    py = points_coords[..., 1]
    b0x = boxes_coords[..., 0]
    b0y = boxes_coords[..., 1]
    b1x = boxes_coords[..., 2]
    b1y = boxes_coords[..., 3]
    plab = points_labels.astype(jnp.int32)
    blab = boxes_labels.astype(jnp.int32)
    bup2 = b_up.reshape(1, 2 * D)

    def batch_spec(n):
        return pl.BlockSpec((bB, n), lambda i: (i, 0))

    def full_spec(shape):
        return pl.BlockSpec(shape, lambda i: tuple(0 for _ in shape))

    out = pl.pallas_call(
        functools.partial(_body, np_=NP, nb=NB, half_d=HALF_D),
        grid=grid,
        in_specs=[
            batch_spec(NP), batch_spec(NP), batch_spec(NP),
            batch_spec(NB), batch_spec(NB), batch_spec(NB), batch_spec(NB),
            batch_spec(NB),
            full_spec((2, HALF_D)),
            full_spec((1, D)), full_spec((1, D)),
            full_spec((1, 2 * D)), full_spec((1, 2 * D)),
            full_spec((1, D)), full_spec((1, D)),
            full_spec((2 * D, D)), full_spec((1, 2 * D)),
        ],
        out_specs=pl.BlockSpec((bB, NP + NB, 2 * D), lambda i: (i, 0, 0)),
        out_shape=jax.ShapeDtypeStruct((B, NP + NB, 2 * D), jnp.float32),
        compiler_params=pltpu.CompilerParams(
            dimension_semantics=("arbitrary",),
        ),
    )(px, py, plab, b0x, b0y, b1x, b1y, blab,
      pe_gauss, pt_w0, pt_w1, bx_w0, bx_w1, bx_w2, bx_w3, W_up, bup2)
    return out


# speedup vs baseline: 3.8572x; 1.2468x over previous
import functools
import math

import jax
import jax.numpy as jnp
from jax.experimental import pallas as pl
from jax.experimental.pallas import tpu as pltpu

_SIN_C = (6.283080178478056, -41.33227074028221, 81.38037233636095,
          -74.54509834593641, 32.89453310276691)
_COS_C = (0.999999660192875, -19.73903217091741, 64.93001305163178,
          -85.28516993938133, 58.84794870978555, -21.15822729330883)


def _sincos_2pi(p):
    k = jax.lax.round(p, jax.lax.RoundingMethod.TO_NEAREST_EVEN)
    u = p - k
    w = u * u
    s = _SIN_C[-1]
    for c in _SIN_C[-2::-1]:
        s = s * w + c
    s = s * u
    c_ = _COS_C[-1]
    for c in _COS_C[-2::-1]:
        c_ = c_ * w + c
    return s, c_


def _body(px_ref, py_ref, plab_ref,
          b0x_ref, b0y_ref, b1x_ref, b1y_ref, blab_ref,
          gauss_ref, ptw0_ref, ptw1_ref,
          bxw0_ref, bxw1_ref, bxw2_ref, bxw3_ref,
          wup_ref, bup_ref, out_ref, *, np_, nb, half_d):
    def _bf(v):
        return v.astype(jnp.bfloat16).astype(jnp.float32)

    g = gauss_ref[...]
    g0 = _bf(g[0:1, :][None])
    g1 = _bf(g[1:2, :][None])

    def _proj(x, y):
        tx = _bf(2.0 * (x + 0.5) - 1.0)[:, :, None]
        ty = _bf(2.0 * (y + 0.5) - 1.0)[:, :, None]
        return tx * g0 + ty * g1

    s, c = _sincos_2pi(_proj(px_ref[...], py_ref[...]))
    emb = jnp.concatenate([s, c], axis=-1)

    lab = plab_ref[...][:, :, None]
    pdw = (ptw1_ref[...] - ptw0_ref[...])[None]
    emb = emb + lab * pdw

    bb = emb.shape[0]
    d = 2 * half_d
    emb2 = emb.reshape(bb * np_, d)
    wup_bf = wup_ref[...].astype(jnp.bfloat16)
    pts = jax.lax.dot_general(emb2.astype(jnp.bfloat16), wup_bf,
                              (((1,), (1,)), ((), ())),
                              preferred_element_type=jnp.float32)
    bias = bup_ref[...] + jax.lax.dot_general(
        ptw0_ref[...].astype(jnp.bfloat16), wup_bf,
        (((1,), (1,)), ((), ())), preferred_element_type=jnp.float32)
    pts = pts + bias
    out_ref[:, 0:np_, :] = pts.reshape(bb, np_, 2 * d)

    s0, c0 = _sincos_2pi(_proj(b0x_ref[...], b0y_ref[...]))
    s1, c1 = _sincos_2pi(_proj(b1x_ref[...], b1y_ref[...]))
    box = jnp.concatenate([s0, c0, s1, c1], axis=-1)
    corner_row = jnp.concatenate([bxw2_ref[...], bxw3_ref[...]], axis=-1)
    crow = (corner_row + bxw0_ref[...])[None]
    bdw = (bxw1_ref[...] - bxw0_ref[...])[None]
    blab = blab_ref[...][:, :, None]
    box = box + (crow + blab * bdw)
    out_ref[:, np_:np_ + nb, :] = box


def kernel(points_coords, points_labels, boxes_coords, boxes_labels,
           pe_gauss, pt_w0, pt_w1, bx_w0, bx_w1, bx_w2, bx_w3, W_up, b_up):
    B, NP, _ = points_coords.shape
    NB = boxes_coords.shape[1]
    HALF_D = pe_gauss.shape[1]
    D = 2 * HALF_D

    bB = 64
    grid = (B // bB,)

    px = points_coords[..., 0]
    py = points_coords[..., 1]
    b0x = boxes_coords[..., 0]
    b0y = boxes_coords[..., 1]
    b1x = boxes_coords[..., 2]
    b1y = boxes_coords[..., 3]
    plab = points_labels.astype(jnp.float32)
    blab = boxes_labels.astype(jnp.float32)
    bup2 = b_up.reshape(1, 2 * D)

    def batch_spec(n):
        return pl.BlockSpec((bB, n), lambda i: (i, 0))

    def full_spec(shape):
        return pl.BlockSpec(shape, lambda i: tuple(0 for _ in shape))

    out = pl.pallas_call(
        functools.partial(_body, np_=NP, nb=NB, half_d=HALF_D),
        grid=grid,
        in_specs=[
            batch_spec(NP), batch_spec(NP), batch_spec(NP),
            batch_spec(NB), batch_spec(NB), batch_spec(NB), batch_spec(NB),
            batch_spec(NB),
            full_spec((2, HALF_D)),
            full_spec((1, D)), full_spec((1, D)),
            full_spec((1, 2 * D)), full_spec((1, 2 * D)),
            full_spec((1, D)), full_spec((1, D)),
            full_spec((2 * D, D)), full_spec((1, 2 * D)),
        ],
        out_specs=pl.BlockSpec((bB, NP + NB, 2 * D), lambda i: (i, 0, 0)),
        out_shape=jax.ShapeDtypeStruct((B, NP + NB, 2 * D), jnp.float32),
        compiler_params=pltpu.CompilerParams(
            dimension_semantics=("arbitrary",),
        ),
    )(px, py, plab, b0x, b0y, b1x, b1y, blab,
      pe_gauss, pt_w0, pt_w1, bx_w0, bx_w1, bx_w2, bx_w3, W_up, bup2)
    return out


# speedup vs baseline: 3.8879x; 1.0080x over previous
import functools
import math

import jax
import jax.numpy as jnp
from jax.experimental import pallas as pl
from jax.experimental.pallas import tpu as pltpu

_SIN_C = (6.279329290982837, -41.11188893993706, 78.06081513301493,
          -56.36503228267863)
_COS_C = (0.9999814280910592, -19.73258915512892, 64.69856260154609,
          -82.54685792407776, 45.91249533049645)


def _sincos_2pi(p):
    k = jax.lax.round(p, jax.lax.RoundingMethod.TO_NEAREST_EVEN)
    u = p - k
    w = u * u
    s = _SIN_C[-1]
    for c in _SIN_C[-2::-1]:
        s = s * w + c
    s = s * u
    c_ = _COS_C[-1]
    for c in _COS_C[-2::-1]:
        c_ = c_ * w + c
    return s, c_


def _body(px_ref, py_ref, plab_ref,
          b0x_ref, b0y_ref, b1x_ref, b1y_ref, blab_ref,
          gauss_ref, ptw0_ref, ptw1_ref,
          bxw0_ref, bxw1_ref, bxw2_ref, bxw3_ref,
          wup_ref, bup_ref, out_ref, *, np_, nb, half_d):
    def _bf(v):
        return v.astype(jnp.bfloat16).astype(jnp.float32)

    g = gauss_ref[...]
    g0 = _bf(g[0:1, :][None])
    g1 = _bf(g[1:2, :][None])

    def _proj(x, y):
        tx = _bf(2.0 * (x + 0.5) - 1.0)[:, :, None]
        ty = _bf(2.0 * (y + 0.5) - 1.0)[:, :, None]
        return tx * g0 + ty * g1

    s, c = _sincos_2pi(_proj(px_ref[...], py_ref[...]))
    emb = jnp.concatenate([s, c], axis=-1)

    lab = plab_ref[...][:, :, None]
    pdw = (ptw1_ref[...] - ptw0_ref[...])[None]
    emb = emb + lab * pdw

    bb = emb.shape[0]
    d = 2 * half_d
    emb2 = emb.reshape(bb * np_, d)
    wup_bf = wup_ref[...].astype(jnp.bfloat16)
    pts = jax.lax.dot_general(emb2.astype(jnp.bfloat16), wup_bf,
                              (((1,), (1,)), ((), ())),
                              preferred_element_type=jnp.float32)
    bias = bup_ref[...] + jax.lax.dot_general(
        ptw0_ref[...].astype(jnp.bfloat16), wup_bf,
        (((1,), (1,)), ((), ())), preferred_element_type=jnp.float32)
    pts = pts + bias
    out_ref[:, 0:np_, :] = pts.reshape(bb, np_, 2 * d)

    s0, c0 = _sincos_2pi(_proj(b0x_ref[...], b0y_ref[...]))
    s1, c1 = _sincos_2pi(_proj(b1x_ref[...], b1y_ref[...]))
    blab = blab_ref[...][:, :, None]
    corner_w = (bxw2_ref[...], bxw2_ref[...], bxw3_ref[...], bxw3_ref[...])
    for q, (piece, cw) in enumerate(zip((s0, c0, s1, c1), corner_w)):
        lo = q * half_d
        crow_q = (cw[:, (q % 2) * half_d:(q % 2 + 1) * half_d]
                  + bxw0_ref[:, lo:lo + half_d])[None]
        bdw_q = (bxw1_ref[:, lo:lo + half_d]
                 - bxw0_ref[:, lo:lo + half_d])[None]
        out_ref[:, np_:np_ + nb, lo:lo + half_d] = (
            piece + (crow_q + blab * bdw_q))


def kernel(points_coords, points_labels, boxes_coords, boxes_labels,
           pe_gauss, pt_w0, pt_w1, bx_w0, bx_w1, bx_w2, bx_w3, W_up, b_up):
    B, NP, _ = points_coords.shape
    NB = boxes_coords.shape[1]
    HALF_D = pe_gauss.shape[1]
    D = 2 * HALF_D

    bB = 128
    grid = (B // bB,)

    px = points_coords[..., 0]
    py = points_coords[..., 1]
    b0x = boxes_coords[..., 0]
    b0y = boxes_coords[..., 1]
    b1x = boxes_coords[..., 2]
    b1y = boxes_coords[..., 3]
    plab = points_labels.astype(jnp.float32)
    blab = boxes_labels.astype(jnp.float32)
    bup2 = b_up.reshape(1, 2 * D)

    def batch_spec(n):
        return pl.BlockSpec((bB, n), lambda i: (i, 0))

    def full_spec(shape):
        return pl.BlockSpec(shape, lambda i: tuple(0 for _ in shape))

    out = pl.pallas_call(
        functools.partial(_body, np_=NP, nb=NB, half_d=HALF_D),
        grid=grid,
        in_specs=[
            batch_spec(NP), batch_spec(NP), batch_spec(NP),
            batch_spec(NB), batch_spec(NB), batch_spec(NB), batch_spec(NB),
            batch_spec(NB),
            full_spec((2, HALF_D)),
            full_spec((1, D)), full_spec((1, D)),
            full_spec((1, 2 * D)), full_spec((1, 2 * D)),
            full_spec((1, D)), full_spec((1, D)),
            full_spec((2 * D, D)), full_spec((1, 2 * D)),
        ],
        out_specs=pl.BlockSpec((bB, NP + NB, 2 * D), lambda i: (i, 0, 0)),
        out_shape=jax.ShapeDtypeStruct((B, NP + NB, 2 * D), jnp.float32),
        compiler_params=pltpu.CompilerParams(
            dimension_semantics=("arbitrary",),
        ),
    )(px, py, plab, b0x, b0y, b1x, b1y, blab,
      pe_gauss, pt_w0, pt_w1, bx_w0, bx_w1, bx_w2, bx_w3, W_up, bup2)
    return out


# speedup vs baseline: 5.4985x; 1.4143x over previous
import functools
import math

import jax
import jax.numpy as jnp
from jax.experimental import pallas as pl
from jax.experimental.pallas import tpu as pltpu

_SIN_C = (6.279329290982837, -41.11188893993706, 78.06081513301493,
          -56.36503228267863)


def _sin_2pi(p):
    k = jax.lax.round(p, jax.lax.RoundingMethod.TO_NEAREST_EVEN)
    u = p - k
    w = u * u
    s = _SIN_C[-1]
    for c in _SIN_C[-2::-1]:
        s = s * w + c
    return s * u


def _body(xall_ref, yall_ref, plab_ref, blab_ref,
          gauss_ref, ptw0_ref, ptw1_ref,
          bxw0_ref, bxw1_ref, bxw2_ref, bxw3_ref,
          wup_ref, bup_ref, out_ref, *, np_, nb, half_d):
    d = 2 * half_d
    f32 = jnp.float32

    def _bf(v):
        return v.astype(jnp.bfloat16).astype(f32)

    g = _bf(gauss_ref[...])
    g0c = jnp.concatenate([g[0:1], g[0:1]], axis=-1)[None]
    g1c = jnp.concatenate([g[1:2], g[1:2]], axis=-1)[None]
    lane = jax.lax.broadcasted_iota(jnp.int32, (1, 1, d), 2)
    off = jnp.where(lane < half_d, 0.0, 0.25).astype(f32)

    xa = _bf(2.0 * (xall_ref[...] + 0.5) - 1.0)[:, :, None]
    ya = _bf(2.0 * (yall_ref[...] + 0.5) - 1.0)[:, :, None]

    p_all = xa * g0c + (ya * g1c + off)
    e_all = _sin_2pi(p_all)

    lab = plab_ref[...][:, :, None]
    pdw = (ptw1_ref[...] - ptw0_ref[...])[None]
    emb = e_all[:, :np_, :] + lab * pdw

    bb = emb.shape[0]
    wup_bf = wup_ref[...].astype(jnp.bfloat16)
    pts = jax.lax.dot_general(emb.reshape(bb * np_, d).astype(jnp.bfloat16),
                              wup_bf, (((1,), (1,)), ((), ())),
                              preferred_element_type=f32)
    bias = bup_ref[...] + jax.lax.dot_general(
        ptw0_ref[...].astype(jnp.bfloat16), wup_bf,
        (((1,), (1,)), ((), ())), preferred_element_type=f32)
    out_ref[:, 0:np_, :] = (pts + bias).reshape(bb, np_, 2 * d)

    blab = blab_ref[...][:, :, None]
    for q, cw_ref in enumerate((bxw2_ref, bxw3_ref)):
        lo = q * d
        crow = (cw_ref[...] + bxw0_ref[:, lo:lo + d])[None]
        bdw = (bxw1_ref[:, lo:lo + d] - bxw0_ref[:, lo:lo + d])[None]
        corner = e_all[:, np_ + q * nb:np_ + (q + 1) * nb, :]
        out_ref[:, np_:np_ + nb, lo:lo + d] = corner + (crow + blab * bdw)


def kernel(points_coords, points_labels, boxes_coords, boxes_labels,
           pe_gauss, pt_w0, pt_w1, bx_w0, bx_w1, bx_w2, bx_w3, W_up, b_up):
    B, NP, _ = points_coords.shape
    NB = boxes_coords.shape[1]
    HALF_D = pe_gauss.shape[1]
    D = 2 * HALF_D

    bB = 128
    grid = (B // bB,)

    xall = jnp.concatenate(
        [points_coords[..., 0], boxes_coords[..., 0], boxes_coords[..., 2]],
        axis=1)
    yall = jnp.concatenate(
        [points_coords[..., 1], boxes_coords[..., 1], boxes_coords[..., 3]],
        axis=1)
    plab = points_labels.astype(jnp.float32)
    blab = boxes_labels.astype(jnp.float32)
    bup2 = b_up.reshape(1, 2 * D)
    nall = NP + 2 * NB

    def batch_spec(n):
        return pl.BlockSpec((bB, n), lambda i: (i, 0))

    def full_spec(shape):
        return pl.BlockSpec(shape, lambda i: tuple(0 for _ in shape))

    out = pl.pallas_call(
        functools.partial(_body, np_=NP, nb=NB, half_d=HALF_D),
        grid=grid,
        in_specs=[
            batch_spec(nall), batch_spec(nall),
            batch_spec(NP), batch_spec(NB),
            full_spec((2, HALF_D)),
            full_spec((1, D)), full_spec((1, D)),
            full_spec((1, 2 * D)), full_spec((1, 2 * D)),
            full_spec((1, D)), full_spec((1, D)),
            full_spec((2 * D, D)), full_spec((1, 2 * D)),
        ],
        out_specs=pl.BlockSpec((bB, NP + NB, 2 * D), lambda i: (i, 0, 0)),
        out_shape=jax.ShapeDtypeStruct((B, NP + NB, 2 * D), jnp.float32),
        compiler_params=pltpu.CompilerParams(
            dimension_semantics=("arbitrary",),
        ),
    )(xall, yall, plab, blab,
      pe_gauss, pt_w0, pt_w1, bx_w0, bx_w1, bx_w2, bx_w3, W_up, bup2)
    return out
